# baseline (speedup 1.0000x reference)
import functools

import jax
import jax.numpy as jnp
from jax.experimental import pallas as pl
from jax.experimental.pallas import tpu as pltpu

S_DIM = 48
V_DIM = 12
H_DIM = 96
N_LAYERS = 3

_LOG2E = 1.4426950408889634


def _silu(x):
    return x / (1.0 + jnp.exp2(x * (-_LOG2E)))


def _egnn_kernel(pos_ref, h0_ref,
                 W3T_ref, AT_ref, BT_ref, We2Ta_ref,
                 WxT_ref, bx_ref,
                 Wh1hT_ref, Wh1aT_ref, Wh1vT_ref, bh1_ref,
                 Wh2T_ref, bh2_ref, WsT_ref, bs_ref,
                 out_ref):
    N = pos_ref.shape[2]
    pos = pos_ref[0]
    centroid = jnp.mean(pos, axis=1, keepdims=True)
    inv = 1.0 / (N - 1)

    rel_cols = []
    mf_cols = []
    for j in range(N):
        pj = pos[:, j:j + 1]
        rel_cols.append(pos - pj)
        midj = 0.5 * (pos + pj) - centroid
        mf_cols.append(jnp.sum(midj * midj, axis=0, keepdims=True))
    relT = jnp.concatenate(rel_cols, axis=1)
    dist2 = jnp.sum(relT * relT, axis=0, keepdims=True)
    midfeat = jnp.concatenate(mf_cols, axis=1)
    ones_row = jnp.ones((1, N * N), jnp.float32)
    ef = jnp.concatenate([dist2, midfeat, ones_row], axis=0)

    md_diag = jnp.sum((pos - centroid) ** 2, axis=0, keepdims=True)
    efd = jnp.concatenate([jnp.zeros((1, N), jnp.float32), md_diag,
                           jnp.ones((1, N), jnp.float32)], axis=0)
    Srel = N * (pos - centroid)

    h = jnp.broadcast_to(h0_ref[...], (S_DIM, N))
    v0 = jnp.zeros((V_DIM, N), jnp.float32)
    v1 = jnp.zeros((V_DIM, N), jnp.float32)
    v2 = jnp.zeros((V_DIM, N), jnp.float32)

    for l in range(N_LAYERS):
        W3T = W3T_ref[l]
        AT = AT_ref[l]
        BT = BT_ref[l]
        We2Ta = We2Ta_ref[l]
        WxT = WxT_ref[l]
        bx = bx_ref[l]

        hA = jnp.dot(AT, h, preferred_element_type=jnp.float32)
        hB = jnp.dot(BT, h, preferred_element_type=jnp.float32)
        rank2 = jnp.dot(W3T, ef, preferred_element_type=jnp.float32)

        pre_d = (hA + hB
                 + jnp.dot(W3T, efd, preferred_element_type=jnp.float32))
        m1d = jnp.concatenate([_silu(pre_d), jnp.ones((1, N), jnp.float32)],
                              axis=0)
        m_d = _silu(jnp.dot(We2Ta, m1d, preferred_element_type=jnp.float32))

        CJ = 16
        agg_s = jnp.zeros((H_DIM, N), jnp.float32)
        G0 = jnp.zeros((H_DIM, N), jnp.float32)
        G1 = jnp.zeros((H_DIM, N), jnp.float32)
        G2 = jnp.zeros((H_DIM, N), jnp.float32)
        for k in range(0, N, CJ):
            cols = []
            for j in range(k, k + CJ):
                pre_j = rank2[:, j * N:(j + 1) * N] + hA + hB[:, j:j + 1]
                cols.append(_silu(pre_j))
            m1_k = jnp.concatenate(cols, axis=1)
            m1_k = jnp.concatenate(
                [m1_k, jnp.ones((1, CJ * N), jnp.float32)], axis=0)
            mm_k = jnp.dot(We2Ta, m1_k,
                           preferred_element_type=jnp.float32)
            for jj in range(CJ):
                sl = slice((k + jj) * N, (k + jj + 1) * N)
                blk = _silu(mm_k[:, jj * N:(jj + 1) * N])
                agg_s = agg_s + blk
                G0 = G0 + relT[0:1, sl] * blk
                G1 = G1 + relT[1:2, sl] * blk
                G2 = G2 + relT[2:3, sl] * blk

        agg = (agg_s - m_d) * inv

        dvs = []
        for c, G in enumerate((G0, G1, G2)):
            dv_c = (jnp.dot(WxT, G, preferred_element_type=jnp.float32)
                    + bx * Srel[c:c + 1, :]) * inv
            dvs.append(dv_c)
        v0 = v0 + dvs[0]
        v1 = v1 + dvs[1]
        v2 = v2 + dvs[2]
        vnorm = v0 * v0 + v1 * v1 + v2 * v2

        u_pre = (jnp.dot(Wh1hT_ref[l], h, preferred_element_type=jnp.float32)
                 + jnp.dot(Wh1aT_ref[l], agg,
                           preferred_element_type=jnp.float32)
                 + jnp.dot(Wh1vT_ref[l], vnorm,
                           preferred_element_type=jnp.float32)
                 + bh1_ref[l])
        u = (jnp.dot(Wh2T_ref[l], _silu(u_pre),
                     preferred_element_type=jnp.float32) + bh2_ref[l])
        h = h + u

    out = jnp.dot(WsT_ref[...], h, preferred_element_type=jnp.float32) \
        + bs_ref[...]
    out_ref[0, 0] = out[0]


@functools.partial(jax.jit, static_argnames=("interpret",))
def _run(positions_t, flat_weights, interpret=False):
    B, _, N = positions_t.shape

    def full(x):
        return pl.BlockSpec(x.shape, lambda b: (0,) * x.ndim)

    in_specs = [pl.BlockSpec((1, 3, N), lambda b: (b, 0, 0))]
    in_specs += [full(x) for x in flat_weights]

    out = pl.pallas_call(
        _egnn_kernel,
        grid=(B,),
        in_specs=in_specs,
        out_specs=pl.BlockSpec((1, 1, N), lambda b: (b, 0, 0)),
        out_shape=jax.ShapeDtypeStruct((B, 1, N), jnp.float32),
        compiler_params=pltpu.CompilerParams(
            dimension_semantics=("parallel",)),
        interpret=interpret,
    )(positions_t, *flat_weights)
    return out.reshape(B, N)


def _flatten_params(params):
    ls = params["layers"]

    def stack(f):
        return jnp.stack([f(l) for l in ls])

    h0 = params["h0"].reshape(S_DIM, 1)
    W3T = stack(lambda l: jnp.concatenate(
        [l["We1"][2 * S_DIM:2 * S_DIM + 2], l["be1"].reshape(1, H_DIM)],
        axis=0).T)
    AT = stack(lambda l: l["We1"][:S_DIM].T)
    BT = stack(lambda l: l["We1"][S_DIM:2 * S_DIM].T)
    We2Ta = stack(lambda l: jnp.concatenate(
        [l["We2"].T, l["be2"].reshape(H_DIM, 1)], axis=1))
    WxT = stack(lambda l: l["Wx"].T)
    bx = stack(lambda l: l["bx"].reshape(V_DIM, 1))
    Wh1hT = stack(lambda l: l["Wh1"][:S_DIM].T)
    Wh1aT = stack(lambda l: l["Wh1"][S_DIM:S_DIM + H_DIM].T)
    Wh1vT = stack(lambda l: l["Wh1"][S_DIM + H_DIM:].T)
    bh1 = stack(lambda l: l["bh1"].reshape(H_DIM, 1))
    Wh2T = stack(lambda l: l["Wh2"].T)
    bh2 = stack(lambda l: l["bh2"].reshape(S_DIM, 1))
    WsT = params["Ws"].T
    bs = params["bs"].reshape(1, 1)
    return (h0, W3T, AT, BT, We2Ta, WxT, bx,
            Wh1hT, Wh1aT, Wh1vT, bh1, Wh2T, bh2, WsT, bs)


def kernel(positions, params):
    return _run(positions.transpose(0, 2, 1), _flatten_params(params))


# speedup vs baseline: 1.1708x; 1.1708x over previous
import functools

import jax
import jax.numpy as jnp
from jax.experimental import pallas as pl
from jax.experimental.pallas import tpu as pltpu

S_DIM = 48
V_DIM = 12
H_DIM = 96
N_LAYERS = 3

def _silu(x):
    y = 0.5 * x
    return y + y * jnp.tanh(y)


def _egnn_kernel(pos_ref, h0_ref,
                 W3T_ref, AT_ref, BT_ref, We2Ta_ref,
                 WxT_ref, bx_ref,
                 Wh1hT_ref, Wh1aT_ref, Wh1vT_ref, bh1_ref,
                 Wh2T_ref, bh2_ref, WsT_ref, bs_ref,
                 out_ref):
    N = pos_ref.shape[2]
    pos = pos_ref[0]
    centroid = jnp.mean(pos, axis=1, keepdims=True)
    inv = 1.0 / (N - 1)

    rel_cols = []
    mf_cols = []
    for j in range(N):
        pj = pos[:, j:j + 1]
        rel_cols.append(pos - pj)
        midj = 0.5 * (pos + pj) - centroid
        mf_cols.append(jnp.sum(midj * midj, axis=0, keepdims=True))
    relT = jnp.concatenate(rel_cols, axis=1)
    dist2 = jnp.sum(relT * relT, axis=0, keepdims=True)
    midfeat = jnp.concatenate(mf_cols, axis=1)
    ones_row = jnp.ones((1, N * N), jnp.float32)
    ef = jnp.concatenate([dist2, midfeat, ones_row], axis=0)

    md_diag = jnp.sum((pos - centroid) ** 2, axis=0, keepdims=True)
    efd = jnp.concatenate([jnp.zeros((1, N), jnp.float32), md_diag,
                           jnp.ones((1, N), jnp.float32)], axis=0)
    Srel = N * (pos - centroid)

    h = jnp.broadcast_to(h0_ref[...], (S_DIM, N))
    v0 = jnp.zeros((V_DIM, N), jnp.float32)
    v1 = jnp.zeros((V_DIM, N), jnp.float32)
    v2 = jnp.zeros((V_DIM, N), jnp.float32)

    for l in range(N_LAYERS):
        W3T = W3T_ref[l]
        AT = AT_ref[l]
        BT = BT_ref[l]
        We2Ta = We2Ta_ref[l]
        WxT = WxT_ref[l]
        bx = bx_ref[l]

        hA = jnp.dot(AT, h, preferred_element_type=jnp.float32)
        hB = jnp.dot(BT, h, preferred_element_type=jnp.float32)
        rank2 = jnp.dot(W3T, ef, preferred_element_type=jnp.float32)

        m1_cols = []
        for j in range(N):
            pre_j = rank2[:, j * N:(j + 1) * N] + hA + hB[:, j:j + 1]
            m1_cols.append(_silu(pre_j))
        m1 = jnp.concatenate(m1_cols, axis=1)
        m1 = jnp.concatenate([m1, ones_row], axis=0)
        mm = jnp.dot(We2Ta, m1, preferred_element_type=jnp.float32)

        pre_d = (hA + hB
                 + jnp.dot(W3T, efd, preferred_element_type=jnp.float32))
        m1d = jnp.concatenate([_silu(pre_d), jnp.ones((1, N), jnp.float32)],
                              axis=0)
        m_d = _silu(jnp.dot(We2Ta, m1d, preferred_element_type=jnp.float32))

        agg_s = jnp.zeros((H_DIM, N), jnp.float32)
        G0 = jnp.zeros((H_DIM, N), jnp.float32)
        G1 = jnp.zeros((H_DIM, N), jnp.float32)
        G2 = jnp.zeros((H_DIM, N), jnp.float32)
        for j in range(N):
            sl = slice(j * N, (j + 1) * N)
            blk = _silu(mm[:, sl])
            agg_s = agg_s + blk
            G0 = G0 + relT[0:1, sl] * blk
            G1 = G1 + relT[1:2, sl] * blk
            G2 = G2 + relT[2:3, sl] * blk

        agg = (agg_s - m_d) * inv

        dvs = []
        for c, G in enumerate((G0, G1, G2)):
            dv_c = (jnp.dot(WxT, G, preferred_element_type=jnp.float32)
                    + bx * Srel[c:c + 1, :]) * inv
            dvs.append(dv_c)
        v0 = v0 + dvs[0]
        v1 = v1 + dvs[1]
        v2 = v2 + dvs[2]
        vnorm = v0 * v0 + v1 * v1 + v2 * v2

        u_pre = (jnp.dot(Wh1hT_ref[l], h, preferred_element_type=jnp.float32)
                 + jnp.dot(Wh1aT_ref[l], agg,
                           preferred_element_type=jnp.float32)
                 + jnp.dot(Wh1vT_ref[l], vnorm,
                           preferred_element_type=jnp.float32)
                 + bh1_ref[l])
        u = (jnp.dot(Wh2T_ref[l], _silu(u_pre),
                     preferred_element_type=jnp.float32) + bh2_ref[l])
        h = h + u

    out = jnp.dot(WsT_ref[...], h, preferred_element_type=jnp.float32) \
        + bs_ref[...]
    out_ref[0, 0] = out[0]


@functools.partial(jax.jit, static_argnames=("interpret",))
def _run(positions_t, flat_weights, interpret=False):
    B, _, N = positions_t.shape

    def full(x):
        return pl.BlockSpec(x.shape, lambda b: (0,) * x.ndim)

    in_specs = [pl.BlockSpec((1, 3, N), lambda b: (b, 0, 0))]
    in_specs += [full(x) for x in flat_weights]

    out = pl.pallas_call(
        _egnn_kernel,
        grid=(B,),
        in_specs=in_specs,
        out_specs=pl.BlockSpec((1, 1, N), lambda b: (b, 0, 0)),
        out_shape=jax.ShapeDtypeStruct((B, 1, N), jnp.float32),
        compiler_params=pltpu.CompilerParams(
            dimension_semantics=("parallel",)),
        interpret=interpret,
    )(positions_t, *flat_weights)
    return out.reshape(B, N)


def _flatten_params(params):
    ls = params["layers"]

    def stack(f):
        return jnp.stack([f(l) for l in ls])

    h0 = params["h0"].reshape(S_DIM, 1)
    W3T = stack(lambda l: jnp.concatenate(
        [l["We1"][2 * S_DIM:2 * S_DIM + 2], l["be1"].reshape(1, H_DIM)],
        axis=0).T)
    AT = stack(lambda l: l["We1"][:S_DIM].T)
    BT = stack(lambda l: l["We1"][S_DIM:2 * S_DIM].T)
    We2Ta = stack(lambda l: jnp.concatenate(
        [l["We2"].T, l["be2"].reshape(H_DIM, 1)], axis=1))
    WxT = stack(lambda l: l["Wx"].T)
    bx = stack(lambda l: l["bx"].reshape(V_DIM, 1))
    Wh1hT = stack(lambda l: l["Wh1"][:S_DIM].T)
    Wh1aT = stack(lambda l: l["Wh1"][S_DIM:S_DIM + H_DIM].T)
    Wh1vT = stack(lambda l: l["Wh1"][S_DIM + H_DIM:].T)
    bh1 = stack(lambda l: l["bh1"].reshape(H_DIM, 1))
    Wh2T = stack(lambda l: l["Wh2"].T)
    bh2 = stack(lambda l: l["bh2"].reshape(S_DIM, 1))
    WsT = params["Ws"].T
    bs = params["bs"].reshape(1, 1)
    return (h0, W3T, AT, BT, We2Ta, WxT, bx,
            Wh1hT, Wh1aT, Wh1vT, bh1, Wh2T, bh2, WsT, bs)


def kernel(positions, params):
    return _run(positions.transpose(0, 2, 1), _flatten_params(params))
